# Initial kernel scaffold; baseline (speedup 1.0000x reference)
import jax
import jax.numpy as jnp
from jax.experimental import pallas as pl


def kernel(src_word, src_pos, emb0_table, emb1_table):
    raise NotImplementedError("write your pallas kernel here")


# speedup vs baseline: 2.0014x; 2.0014x over previous
import functools

import jax
import jax.numpy as jnp
from jax import lax
from jax.experimental import pallas as pl
from jax.experimental.pallas import tpu as pltpu
from jax.experimental.pallas import tpu_sc as plsc

D = 64
SCALE = 8.0
G = 64
C = 512


@functools.lru_cache(maxsize=None)
def _make_kernel(N):
    NC, NS = 2, 16
    NW = NC * NS
    assert N % (NW * C) == 0
    n_w = N // NW
    n_chunks = n_w // C
    n_g = C // G

    mesh = plsc.VectorSubcoreMesh(
        core_axis_name="c", subcore_axis_name="s", num_cores=NC, num_subcores=NS)

    @functools.partial(
        pl.kernel,
        out_type=jax.ShapeDtypeStruct((N, D), jnp.float32),
        mesh=mesh,
        scratch_types=[
            pltpu.VMEM((n_g, G), jnp.int32),
            pltpu.VMEM((n_g, G), jnp.int32),
            pltpu.VMEM((C, D), jnp.float32),
            pltpu.VMEM((C, D), jnp.float32),
            pltpu.SemaphoreType.DMA,
        ],
        compiler_params=pltpu.CompilerParams(use_tc_tiling_on_sc=False),
    )
    def body(w_hbm, p_hbm, emb0_hbm, emb1_hbm, out_hbm, wi_v, pi_v, wrow_v, orow_v, sem):
        wid = lax.axis_index("s") * NC + lax.axis_index("c")
        row0 = wid * n_w

        def chunk_body(k, carry):
            off = pl.multiple_of(row0 + k * C, C)
            goff = pl.multiple_of(off // G, n_g)
            pltpu.sync_copy(w_hbm.at[pl.ds(goff, n_g)], wi_v)
            pltpu.sync_copy(p_hbm.at[pl.ds(goff, n_g)], pi_v)
            waits = []
            for j in range(n_g):
                waits.append(pltpu.async_copy(
                    emb1_hbm.at[pi_v.at[j]], orow_v.at[pl.ds(j * G, G)], sem))
                waits.append(pltpu.async_copy(
                    emb0_hbm.at[wi_v.at[j]], wrow_v.at[pl.ds(j * G, G)], sem))
            for w in waits:
                w.wait()

            def row_body(r, carry2):
                for c in range(D // 16):
                    sl = pl.ds(c * 16, 16)
                    plsc.addupdate(orow_v.at[r, sl], wrow_v[r, sl] * SCALE)
                return carry2

            lax.fori_loop(0, C, row_body, 0, unroll=2)

            pltpu.sync_copy(orow_v, out_hbm.at[pl.ds(off, C)])
            return carry

        lax.fori_loop(0, n_chunks, chunk_body, 0)

    return body


def kernel(src_word, src_pos, emb0_table, emb1_table):
    B, L, _ = src_word.shape
    N = B * L
    w_flat = jnp.reshape(src_word.astype(jnp.int32), (N // G, G))
    p_flat = jnp.reshape(src_pos.astype(jnp.int32), (N // G, G))
    out = _make_kernel(N)(w_flat, p_flat, emb0_table, emb1_table)
    return jnp.reshape(out, (B, L, D))


# speedup vs baseline: 2.0119x; 1.0052x over previous
import functools

import jax
import jax.numpy as jnp
from jax import lax
from jax.experimental import pallas as pl
from jax.experimental.pallas import tpu as pltpu
from jax.experimental.pallas import tpu_sc as plsc

D = 64
SCALE = 8.0
NB = 4
SEGS = ((0, 104), (104, 96))


@functools.lru_cache(maxsize=None)
def _make_kernel(B, L):
    NC, NS = 2, 16
    NW = NC * NS
    assert B % (NW * NB) == 0
    b_w = B // NW
    n_chunks = b_w // NB

    mesh = plsc.VectorSubcoreMesh(
        core_axis_name="c", subcore_axis_name="s", num_cores=NC, num_subcores=NS)

    @functools.partial(
        pl.kernel,
        out_type=jax.ShapeDtypeStruct((B, L, D), jnp.float32),
        mesh=mesh,
        scratch_types=[
            pltpu.VMEM((NB, L), jnp.int32),
            pltpu.VMEM((NB, L), jnp.int32),
            pltpu.VMEM((NB, L, D), jnp.float32),
            pltpu.VMEM((NB, L, D), jnp.float32),
            pltpu.SemaphoreType.DMA,
        ],
        compiler_params=pltpu.CompilerParams(use_tc_tiling_on_sc=False),
    )
    def body(w_hbm, p_hbm, emb0_hbm, emb1_hbm, out_hbm, wi_v, pi_v, wrow_v, orow_v, sem):
        wid = lax.axis_index("s") * NC + lax.axis_index("c")
        brow0 = wid * b_w

        def chunk_body(k, carry):
            b0 = brow0 + k * NB
            pltpu.sync_copy(w_hbm.at[pl.ds(b0, NB)], wi_v)
            pltpu.sync_copy(p_hbm.at[pl.ds(b0, NB)], pi_v)
            waits = []
            for i in range(NB):
                for (l0, g) in SEGS:
                    sl = pl.ds(l0, g)
                    waits.append(pltpu.async_copy(
                        emb1_hbm.at[pi_v.at[i, sl]], orow_v.at[i, sl], sem))
                    waits.append(pltpu.async_copy(
                        emb0_hbm.at[wi_v.at[i, sl]], wrow_v.at[i, sl], sem))
            for w in waits:
                w.wait()

            def l_body(l, carry2):
                for i in range(NB):
                    for c in range(D // 16):
                        sl = pl.ds(c * 16, 16)
                        plsc.addupdate(orow_v.at[i, l, sl],
                                       wrow_v[i, l, sl] * SCALE)
                return carry2

            lax.fori_loop(0, L, l_body, 0)

            pltpu.sync_copy(orow_v, out_hbm.at[pl.ds(b0, NB)])
            return carry

        lax.fori_loop(0, n_chunks, chunk_body, 0)

    return body


def kernel(src_word, src_pos, emb0_table, emb1_table):
    B, L, _ = src_word.shape
    w = jnp.squeeze(src_word.astype(jnp.int32), axis=-1)
    p = jnp.squeeze(src_pos.astype(jnp.int32), axis=-1)
    return _make_kernel(B, L)(w, p, emb0_table, emb1_table)
